# baseline (speedup 1.0000x reference)
import functools

import jax
import jax.numpy as jnp
from jax import lax
from jax.experimental import pallas as pl
from jax.experimental.pallas import tpu as pltpu
from jax.experimental.pallas import tpu_sc as plsc

VOC = 100000
D = 64
N_MEM = 200
T_Q = 20
T_M = 50
N_HOPS = 3
L = 16
NC = 2
NS = 16
NW = NC * NS
SLOTS_PER_W = 8
N_STORY_W = N_MEM // SLOTS_PER_W
QUERY_W = N_STORY_W


def _slot_sum(rows_ref, row_base, out_ref, out_row, n_rows):
    zero = jnp.zeros((L,), jnp.float32)

    def body(r, accs):
        return tuple(
            accs[c] + rows_ref[row_base + r, pl.ds(c * L, L)]
            for c in range(D // L)
        )

    accs = lax.fori_loop(0, n_rows, body, (zero,) * (D // L))
    for c in range(D // L):
        out_ref[out_row, pl.ds(c * L, L)] = accs[c]


def _sc_body(story_hbm, query_hbm, wa_hbm, wc_hbm, wb_hbm,
             mem_in_hbm, mem_out_hbm, u0_hbm,
             idx_v, qidx_v, rows_a, rows_c, qrows_v,
             acc_in, acc_out, u0_v, sem):
    wid = lax.axis_index("c") * NS + lax.axis_index("s")

    @pl.when(wid < N_STORY_W)
    def _story_work():
        base = wid * SLOTS_PER_W
        pltpu.sync_copy(story_hbm.at[pl.ds(base, SLOTS_PER_W)], idx_v)
        copies = []
        for j in range(SLOTS_PER_W):
            copies.append(pltpu.async_copy(
                wa_hbm.at[idx_v.at[j]], rows_a.at[pl.ds(j * T_M, T_M)], sem))
            copies.append(pltpu.async_copy(
                wc_hbm.at[idx_v.at[j]], rows_c.at[pl.ds(j * T_M, T_M)], sem))
        for cp in copies:
            cp.wait()
        for j in range(SLOTS_PER_W):
            _slot_sum(rows_a, j * T_M, acc_in, j, T_M)
            _slot_sum(rows_c, j * T_M, acc_out, j, T_M)
        pltpu.sync_copy(acc_in, mem_in_hbm.at[pl.ds(base, SLOTS_PER_W)])
        pltpu.sync_copy(acc_out, mem_out_hbm.at[pl.ds(base, SLOTS_PER_W)])

    @pl.when(wid == QUERY_W)
    def _query_work():
        pltpu.sync_copy(query_hbm, qidx_v)
        pltpu.async_copy(wb_hbm.at[qidx_v], qrows_v, sem).wait()
        _slot_sum(qrows_v, 0, u0_v, 0, T_Q)
        pltpu.sync_copy(u0_v, u0_hbm)


_sc_gather_sums = functools.partial(
    pl.kernel,
    out_type=[
        jax.ShapeDtypeStruct((N_MEM, D), jnp.float32),
        jax.ShapeDtypeStruct((N_MEM, D), jnp.float32),
        jax.ShapeDtypeStruct((1, D), jnp.float32),
    ],
    mesh=plsc.VectorSubcoreMesh(core_axis_name="c", subcore_axis_name="s"),
    compiler_params=pltpu.CompilerParams(use_tc_tiling_on_sc=False),
    scratch_types=[
        pltpu.VMEM((SLOTS_PER_W, T_M), jnp.int32),
        pltpu.VMEM((T_Q,), jnp.int32),
        pltpu.VMEM((SLOTS_PER_W * T_M, D), jnp.float32),
        pltpu.VMEM((SLOTS_PER_W * T_M, D), jnp.float32),
        pltpu.VMEM((T_Q, D), jnp.float32),
        pltpu.VMEM((SLOTS_PER_W, D), jnp.float32),
        pltpu.VMEM((SLOTS_PER_W, D), jnp.float32),
        pltpu.VMEM((1, D), jnp.float32),
        pltpu.SemaphoreType.DMA,
    ],
)(_sc_body)


def _tc_body(mem_in_ref, mem_out_ref, u0_ref, ta_ref, tc_ref,
             hw_ref, hb_ref, wout_ref, out_ref):
    mem_in = mem_in_ref[...] + ta_ref[...]
    mem_out = mem_out_ref[...] + tc_ref[...]
    u = u0_ref[...]
    hw = hw_ref[...]
    hb = hb_ref[...]
    for _ in range(N_HOPS):
        attn = lax.dot_general(u, mem_in, (((1,), (1,)), ((), ())),
                               preferred_element_type=jnp.float32)
        attn = attn - jnp.max(attn, axis=1, keepdims=True)
        e = jnp.exp(attn)
        p = e / jnp.sum(e, axis=1, keepdims=True)
        wo = lax.dot_general(p, mem_out, (((1,), (0,)), ((), ())),
                             preferred_element_type=jnp.float32)
        u = u + lax.dot_general(wo, hw, (((1,), (1,)), ((), ())),
                                preferred_element_type=jnp.float32) + hb
    logits = lax.dot_general(u, wout_ref[...], (((1,), (1,)), ((), ())),
                             preferred_element_type=jnp.float32)
    mx = jnp.max(logits, axis=1, keepdims=True)
    lse = mx + jnp.log(jnp.sum(jnp.exp(logits - mx), axis=1, keepdims=True))
    out_ref[...] = logits - lse


def _tc_finish(mem_in, mem_out, u0, TA, TC_pos, H_w, H_b2, weight_out):
    return pl.pallas_call(
        _tc_body,
        out_shape=jax.ShapeDtypeStruct((1, VOC), jnp.float32),
    )(mem_in, mem_out, u0, TA, TC_pos, H_w, H_b2, weight_out)


def kernel(query, story, Wa, Wc, Wb, weight_out, H_w, H_b, TA, TC):
    q = query.reshape(-1).astype(jnp.int32)
    st = story.astype(jnp.int32)
    mem_in, mem_out, u0 = _sc_gather_sums(st, q, Wa, Wc, Wb)
    return _tc_finish(mem_in, mem_out, u0, TA, TC, H_w,
                      H_b.reshape(1, D), weight_out)


# speedup vs baseline: 1.0028x; 1.0028x over previous
import functools

import jax
import jax.numpy as jnp
from jax import lax
from jax.experimental import pallas as pl
from jax.experimental.pallas import tpu as pltpu
from jax.experimental.pallas import tpu_sc as plsc

VOC = 100000
D = 64
N_MEM = 200
T_Q = 20
T_M = 50
N_HOPS = 3
L = 16
NC = 2
NS = 16
NW = NC * NS
SLOTS_PER_W = 8
N_STORY_W = N_MEM // SLOTS_PER_W


def _slot_sum(rows_ref, row_base, out_ref, out_row, n_rows):
    zero = jnp.zeros((L,), jnp.float32)

    def body(r, accs):
        return tuple(
            accs[c] + rows_ref[row_base + r, pl.ds(c * L, L)]
            for c in range(D // L)
        )

    accs = lax.fori_loop(0, n_rows, body, (zero,) * (D // L))
    for c in range(D // L):
        out_ref[out_row, pl.ds(c * L, L)] = accs[c]


def _sc_body(story_hbm, query_hbm, wa_hbm, wc_hbm, wb_hbm,
             mem_in_hbm, mem_out_hbm, u0_hbm,
             idx_v, rows_a, rows_c, acc_in, acc_out,
             qidx, qrows, uacc, sem):
    wid = lax.axis_index("c") * NS + lax.axis_index("s")

    @pl.when(wid < N_STORY_W)
    def _story_work():
        base = wid * SLOTS_PER_W
        pltpu.sync_copy(story_hbm.at[pl.ds(base, SLOTS_PER_W)], idx_v)
        copies = []
        for j in range(SLOTS_PER_W):
            copies.append(pltpu.async_copy(
                wa_hbm.at[idx_v.at[j]], rows_a.at[pl.ds(j * T_M, T_M)], sem))
            copies.append(pltpu.async_copy(
                wc_hbm.at[idx_v.at[j]], rows_c.at[pl.ds(j * T_M, T_M)], sem))
        for cp in copies:
            cp.wait()
        for j in range(SLOTS_PER_W):
            _slot_sum(rows_a, j * T_M, acc_in, j, T_M)
            _slot_sum(rows_c, j * T_M, acc_out, j, T_M)
        pltpu.sync_copy(acc_in, mem_in_hbm.at[pl.ds(base, SLOTS_PER_W)])
        pltpu.sync_copy(acc_out, mem_out_hbm.at[pl.ds(base, SLOTS_PER_W)])

    @pl.when(wid == N_STORY_W)
    def _query_work():
        pltpu.sync_copy(query_hbm, qidx)
        cp = pltpu.async_copy(wb_hbm.at[qidx.at[0]], qrows, sem)
        cp.wait()
        _slot_sum(qrows, 0, uacc, 0, T_Q)
        pltpu.sync_copy(uacc, u0_hbm)


_sc_gather_sums = functools.partial(
    pl.kernel,
    out_type=[
        jax.ShapeDtypeStruct((N_MEM, D), jnp.float32),
        jax.ShapeDtypeStruct((N_MEM, D), jnp.float32),
        jax.ShapeDtypeStruct((1, D), jnp.float32),
    ],
    mesh=plsc.VectorSubcoreMesh(core_axis_name="c", subcore_axis_name="s"),
    compiler_params=pltpu.CompilerParams(use_tc_tiling_on_sc=False),
    scratch_types=[
        pltpu.VMEM((SLOTS_PER_W, T_M), jnp.int32),
        pltpu.VMEM((SLOTS_PER_W * T_M, D), jnp.float32),
        pltpu.VMEM((SLOTS_PER_W * T_M, D), jnp.float32),
        pltpu.VMEM((SLOTS_PER_W, D), jnp.float32),
        pltpu.VMEM((SLOTS_PER_W, D), jnp.float32),
        pltpu.VMEM((1, T_Q), jnp.int32),
        pltpu.VMEM((T_Q, D), jnp.float32),
        pltpu.VMEM((1, D), jnp.float32),
        pltpu.SemaphoreType.DMA,
    ],
)(_sc_body)


def _tc_body(u0_ref, mem_in_ref, mem_out_ref, ta_ref, tc_ref,
             hw_ref, hb_ref, wo_ref, out_ref):
    u = u0_ref[...]
    mem_in = mem_in_ref[...] + ta_ref[...]
    mem_out = mem_out_ref[...] + tc_ref[...]
    hw = hw_ref[...]
    hb = hb_ref[...]
    for _ in range(N_HOPS):
        attn = lax.dot_general(mem_in, u, (((1,), (1,)), ((), ())),
                               preferred_element_type=jnp.float32)
        attn = attn - jnp.max(attn, axis=0, keepdims=True)
        e = jnp.exp(attn)
        p = e / jnp.sum(e, axis=0, keepdims=True)
        wrow = lax.dot_general(p, mem_out, (((0,), (0,)), ((), ())),
                               preferred_element_type=jnp.float32)
        u = u + lax.dot_general(wrow, hw, (((1,), (1,)), ((), ())),
                                preferred_element_type=jnp.float32) + hb
    logits = lax.dot_general(u, wo_ref[...], (((1,), (1,)), ((), ())),
                             preferred_element_type=jnp.float32)
    mx = jnp.max(logits, axis=1, keepdims=True)
    lse = mx + jnp.log(jnp.sum(jnp.exp(logits - mx), axis=1, keepdims=True))
    out_ref[...] = logits - lse


def _tc_finish(u0, mem_in, mem_out, TA, TC_pos, H_w, H_b_row, weight_out):
    return pl.pallas_call(
        _tc_body,
        out_shape=jax.ShapeDtypeStruct((1, VOC), jnp.float32),
        in_specs=[
            pl.BlockSpec(memory_space=pltpu.VMEM),
            pl.BlockSpec(memory_space=pltpu.VMEM),
            pl.BlockSpec(memory_space=pltpu.VMEM),
            pl.BlockSpec(memory_space=pltpu.VMEM),
            pl.BlockSpec(memory_space=pltpu.VMEM),
            pl.BlockSpec(memory_space=pltpu.VMEM),
            pl.BlockSpec(memory_space=pltpu.VMEM),
            pl.BlockSpec(memory_space=pltpu.VMEM),
        ],
    )(u0, mem_in, mem_out, TA, TC_pos, H_w, H_b_row, weight_out)


def kernel(query, story, Wa, Wc, Wb, weight_out, H_w, H_b, TA, TC):
    st = story.astype(jnp.int32)
    q = query.astype(jnp.int32)
    mem_in, mem_out, u0 = _sc_gather_sums(st, q, Wa, Wc, Wb)
    return _tc_finish(u0, mem_in, mem_out, TA, TC, H_w,
                      H_b.reshape(1, D), weight_out)


# speedup vs baseline: 1.0705x; 1.0675x over previous
import functools

import jax
import jax.numpy as jnp
from jax import lax
from jax.experimental import pallas as pl
from jax.experimental.pallas import tpu as pltpu
from jax.experimental.pallas import tpu_sc as plsc

VOC = 100000
D = 64
N_MEM = 200
T_Q = 20
T_M = 50
N_HOPS = 3
L = 16
NC = 2
NS = 16
NW = NC * NS
SLOTS_PER_W = 8
N_STORY_W = N_MEM // SLOTS_PER_W


def _slot_sum(rows_ref, row_base, out_ref, out_row, n_rows):
    zero = jnp.zeros((L,), jnp.float32)

    def body(r, accs):
        return tuple(
            accs[c] + rows_ref[row_base + r, pl.ds(c * L, L)]
            for c in range(D // L)
        )

    accs = lax.fori_loop(0, n_rows, body, (zero,) * (D // L))
    for c in range(D // L):
        out_ref[out_row, pl.ds(c * L, L)] = accs[c]


def _sc_body(story_hbm, wa_hbm, wc_hbm,
             mem_in_hbm, mem_out_hbm,
             idx_v, rows_a, rows_c, acc_in, acc_out, sem):
    wid = lax.axis_index("c") * NS + lax.axis_index("s")

    @pl.when(wid < N_STORY_W)
    def _story_work():
        base = wid * SLOTS_PER_W
        pltpu.sync_copy(story_hbm.at[pl.ds(base, SLOTS_PER_W)], idx_v)
        copies = []
        for j in range(SLOTS_PER_W):
            copies.append(pltpu.async_copy(
                wa_hbm.at[idx_v.at[j]], rows_a.at[pl.ds(j * T_M, T_M)], sem))
            copies.append(pltpu.async_copy(
                wc_hbm.at[idx_v.at[j]], rows_c.at[pl.ds(j * T_M, T_M)], sem))
        for cp in copies:
            cp.wait()
        for j in range(SLOTS_PER_W):
            _slot_sum(rows_a, j * T_M, acc_in, j, T_M)
            _slot_sum(rows_c, j * T_M, acc_out, j, T_M)
        pltpu.sync_copy(acc_in, mem_in_hbm.at[pl.ds(base, SLOTS_PER_W)])
        pltpu.sync_copy(acc_out, mem_out_hbm.at[pl.ds(base, SLOTS_PER_W)])


_sc_gather_sums = functools.partial(
    pl.kernel,
    out_type=[
        jax.ShapeDtypeStruct((N_MEM, D), jnp.float32),
        jax.ShapeDtypeStruct((N_MEM, D), jnp.float32),
    ],
    mesh=plsc.VectorSubcoreMesh(core_axis_name="c", subcore_axis_name="s"),
    compiler_params=pltpu.CompilerParams(use_tc_tiling_on_sc=False),
    scratch_types=[
        pltpu.VMEM((SLOTS_PER_W, T_M), jnp.int32),
        pltpu.VMEM((SLOTS_PER_W * T_M, D), jnp.float32),
        pltpu.VMEM((SLOTS_PER_W * T_M, D), jnp.float32),
        pltpu.VMEM((SLOTS_PER_W, D), jnp.float32),
        pltpu.VMEM((SLOTS_PER_W, D), jnp.float32),
        pltpu.SemaphoreType.DMA,
    ],
)(_sc_body)


def _tc_body(query_smem, mem_in_ref, mem_out_ref, ta_ref, tc_ref,
             hw_ref, hb_ref, wo_ref, wb_hbm, out_ref, qblk_ref, sem):
    copies = []
    for t in range(T_Q):
        q = query_smem[0, t]
        start = pl.multiple_of((q // 8) * 8, 8)
        copies.append(pltpu.make_async_copy(
            wb_hbm.at[pl.ds(start, 8)], qblk_ref.at[pl.ds(t * 8, 8)], sem))
    for cp in copies:
        cp.start()
    for cp in copies:
        cp.wait()
    rid = lax.broadcasted_iota(jnp.int32, (T_Q * 8, 1), 0)
    mask = jnp.zeros((T_Q * 8, 1), jnp.float32)
    for t in range(T_Q):
        qmod = lax.rem(query_smem[0, t], 8)
        mask = mask + jnp.where(rid == t * 8 + qmod, 1.0, 0.0)
    u = lax.dot_general(mask, qblk_ref[...], (((0,), (0,)), ((), ())),
                        preferred_element_type=jnp.float32)

    mem_in = mem_in_ref[...] + ta_ref[...]
    mem_out = mem_out_ref[...] + tc_ref[...]
    hw = hw_ref[...]
    hb = hb_ref[...]
    for _ in range(N_HOPS):
        attn = lax.dot_general(mem_in, u, (((1,), (1,)), ((), ())),
                               preferred_element_type=jnp.float32)
        attn = attn - jnp.max(attn, axis=0, keepdims=True)
        e = jnp.exp(attn)
        p = e / jnp.sum(e, axis=0, keepdims=True)
        wrow = lax.dot_general(p, mem_out, (((0,), (0,)), ((), ())),
                               preferred_element_type=jnp.float32)
        u = u + lax.dot_general(wrow, hw, (((1,), (1,)), ((), ())),
                                preferred_element_type=jnp.float32) + hb
    logits = lax.dot_general(u, wo_ref[...], (((1,), (1,)), ((), ())),
                             preferred_element_type=jnp.float32)
    mx = jnp.max(logits, axis=1, keepdims=True)
    lse = mx + jnp.log(jnp.sum(jnp.exp(logits - mx), axis=1, keepdims=True))
    out_ref[...] = logits - lse


def _tc_finish(query, mem_in, mem_out, TA, TC_pos, H_w, H_b_row, weight_out,
               Wb):
    return pl.pallas_call(
        _tc_body,
        out_shape=jax.ShapeDtypeStruct((1, VOC), jnp.float32),
        in_specs=[
            pl.BlockSpec(memory_space=pltpu.SMEM),
            pl.BlockSpec(memory_space=pltpu.VMEM),
            pl.BlockSpec(memory_space=pltpu.VMEM),
            pl.BlockSpec(memory_space=pltpu.VMEM),
            pl.BlockSpec(memory_space=pltpu.VMEM),
            pl.BlockSpec(memory_space=pltpu.VMEM),
            pl.BlockSpec(memory_space=pltpu.VMEM),
            pl.BlockSpec(memory_space=pltpu.VMEM),
            pl.BlockSpec(memory_space=pl.ANY),
        ],
        scratch_shapes=[
            pltpu.VMEM((T_Q * 8, D), jnp.float32),
            pltpu.SemaphoreType.DMA,
        ],
    )(query, mem_in, mem_out, TA, TC_pos, H_w, H_b_row, weight_out, Wb)


def kernel(query, story, Wa, Wc, Wb, weight_out, H_w, H_b, TA, TC):
    st = story.astype(jnp.int32)
    q = query.astype(jnp.int32)
    mem_in, mem_out = _sc_gather_sums(st, Wa, Wc)
    return _tc_finish(q, mem_in, mem_out, TA, TC, H_w,
                      H_b.reshape(1, D), weight_out, Wb)
